# baseline (speedup 1.0000x reference)
import jax
import jax.numpy as jnp
from jax import lax
from jax.experimental import pallas as pl
from jax.experimental.pallas import tpu as pltpu
from jax.experimental.pallas import tpu_sc as plsc

NUM_EMBEDDINGS = 16 * 1024 * 1024
EMBEDDING_DIM = 4
BATCH = 16384
N_FIELDS = 100

TOTAL = BATCH * N_FIELDS
NUM_WORKERS = 32
PER_WORKER = TOTAL // NUM_WORKERS
GATHER = 128
CHUNK = 12_800
NUM_CHUNKS = PER_WORKER // CHUNK
GATHERS_PER_CHUNK = CHUNK // GATHER
LANES = 16


def _sc_body(idx_hbm, table8_hbm, buf_hbm, idx_v, sidx_v, rows_v, sem):
    wid = lax.axis_index("s") * 2 + lax.axis_index("c")
    base_w = wid * PER_WORKER

    def chunk_step(c, carry):
        base = base_w + c * CHUNK
        pltpu.sync_copy(idx_hbm.at[pl.ds(base, CHUNK)], idx_v)

        def shift(i, c2):
            sidx_v[pl.ds(i * LANES, LANES)] = lax.shift_right_logical(
                idx_v[pl.ds(i * LANES, LANES)], 1
            )
            return c2

        lax.fori_loop(0, CHUNK // LANES, shift, 0)

        def fire(j, c2):
            pltpu.async_copy(
                table8_hbm.at[sidx_v.at[pl.ds(j * GATHER, GATHER)]],
                rows_v.at[pl.ds(j * GATHER, GATHER)],
                sem,
            )
            return c2

        lax.fori_loop(0, GATHERS_PER_CHUNK, fire, 0)

        def drain(j, c2):
            pltpu.make_async_copy(
                table8_hbm.at[sidx_v.at[pl.ds(j * GATHER, GATHER)]],
                rows_v.at[pl.ds(j * GATHER, GATHER)],
                sem,
            ).wait()
            return c2

        lax.fori_loop(0, GATHERS_PER_CHUNK, drain, 0)
        pltpu.sync_copy(rows_v, buf_hbm.at[pl.ds(base, CHUNK)])
        return carry

    lax.fori_loop(0, NUM_CHUNKS, chunk_step, 0)


def _sc_gather(idx_flat, table8):
    mesh = plsc.VectorSubcoreMesh(core_axis_name="c", subcore_axis_name="s")
    return pl.kernel(
        _sc_body,
        out_type=jax.ShapeDtypeStruct((TOTAL, 2 * EMBEDDING_DIM), jnp.float32),
        mesh=mesh,
        scratch_types=[
            pltpu.VMEM((CHUNK,), jnp.int32),
            pltpu.VMEM((CHUNK,), jnp.int32),
            pltpu.VMEM((CHUNK, 2 * EMBEDDING_DIM), jnp.float32),
            pltpu.SemaphoreType.DMA,
        ],
        compiler_params=pltpu.CompilerParams(use_tc_tiling_on_sc=False),
    )(idx_flat, table8)


TC_BLOCK = 12_800
TC_GRID = TOTAL // TC_BLOCK


def _tc_body(idx_ref, buf_ref, out_ref):
    idx = idx_ref[0, 0, :]
    par = (idx & 1).reshape(TC_BLOCK, 1) == 1
    lo = buf_ref[:, 0:EMBEDDING_DIM]
    hi = buf_ref[:, EMBEDDING_DIM : 2 * EMBEDDING_DIM]
    out_ref[...] = jnp.where(par, hi, lo)


def _tc_select(idx_flat, buf):
    idx3 = idx_flat.reshape(TC_GRID, 1, TC_BLOCK)
    return pl.pallas_call(
        _tc_body,
        grid=(TC_GRID,),
        in_specs=[
            pl.BlockSpec((1, 1, TC_BLOCK), lambda i: (i, 0, 0)),
            pl.BlockSpec((TC_BLOCK, 2 * EMBEDDING_DIM), lambda i: (i, 0)),
        ],
        out_specs=pl.BlockSpec((TC_BLOCK, EMBEDDING_DIM), lambda i: (i, 0)),
        out_shape=jax.ShapeDtypeStruct((TOTAL, EMBEDDING_DIM), jnp.float32),
    )(idx3, buf)


@jax.jit
def _run(x, table):
    idx_flat = x.reshape(TOTAL)
    table8 = table.reshape(NUM_EMBEDDINGS // 2, 2 * EMBEDDING_DIM)
    buf = _sc_gather(idx_flat, table8)
    out = _tc_select(idx_flat, buf)
    return out.reshape(BATCH, N_FIELDS, EMBEDDING_DIM)


def kernel(x, table):
    return _run(x, table)


# speedup vs baseline: 1.0004x; 1.0004x over previous
import jax
import jax.numpy as jnp
from jax import lax
from jax.experimental import pallas as pl
from jax.experimental.pallas import tpu as pltpu
from jax.experimental.pallas import tpu_sc as plsc

NUM_EMBEDDINGS = 16 * 1024 * 1024
EMBEDDING_DIM = 4
BATCH = 16384
N_FIELDS = 100

TOTAL = BATCH * N_FIELDS
NUM_WORKERS = 32
PER_WORKER = TOTAL // NUM_WORKERS
GATHER = 128
CHUNK = 12_800
NUM_CHUNKS = PER_WORKER // CHUNK
GATHERS_PER_CHUNK = CHUNK // GATHER
LANES = 16


def _sc_body(idx_hbm, table8_hbm, buf_hbm, idx_v, rows_v, sem):
    wid = lax.axis_index("s") * 2 + lax.axis_index("c")
    base_w = wid * PER_WORKER

    def chunk_step(c, carry):
        base = base_w + c * CHUNK
        pltpu.sync_copy(idx_hbm.at[pl.ds(base, CHUNK)], idx_v)

        def shift(i, c2):
            idx_v[pl.ds(i * LANES, LANES)] = lax.shift_right_logical(
                idx_v[pl.ds(i * LANES, LANES)], 1
            )
            return c2

        lax.fori_loop(0, CHUNK // LANES, shift, 0)
        pltpu.async_copy(table8_hbm.at[idx_v], rows_v, sem).wait()
        pltpu.sync_copy(rows_v, buf_hbm.at[pl.ds(base, CHUNK)])
        return carry

    lax.fori_loop(0, NUM_CHUNKS, chunk_step, 0)


def _sc_gather(idx_flat, table8):
    mesh = plsc.VectorSubcoreMesh(core_axis_name="c", subcore_axis_name="s")
    return pl.kernel(
        _sc_body,
        out_type=jax.ShapeDtypeStruct((TOTAL, 2 * EMBEDDING_DIM), jnp.float32),
        mesh=mesh,
        scratch_types=[
            pltpu.VMEM((CHUNK,), jnp.int32),
            pltpu.VMEM((CHUNK, 2 * EMBEDDING_DIM), jnp.float32),
            pltpu.SemaphoreType.DMA,
        ],
        compiler_params=pltpu.CompilerParams(use_tc_tiling_on_sc=False),
    )(idx_flat, table8)


TC_BLOCK = 12_800
TC_GRID = TOTAL // TC_BLOCK


def _tc_body(idx_ref, buf_ref, out_ref):
    idx = idx_ref[0, 0, :]
    par = (idx & 1).reshape(TC_BLOCK, 1) == 1
    lo = buf_ref[:, 0:EMBEDDING_DIM]
    hi = buf_ref[:, EMBEDDING_DIM : 2 * EMBEDDING_DIM]
    out_ref[...] = jnp.where(par, hi, lo)


def _tc_select(idx_flat, buf):
    idx3 = idx_flat.reshape(TC_GRID, 1, TC_BLOCK)
    return pl.pallas_call(
        _tc_body,
        grid=(TC_GRID,),
        in_specs=[
            pl.BlockSpec((1, 1, TC_BLOCK), lambda i: (i, 0, 0)),
            pl.BlockSpec((TC_BLOCK, 2 * EMBEDDING_DIM), lambda i: (i, 0)),
        ],
        out_specs=pl.BlockSpec((TC_BLOCK, EMBEDDING_DIM), lambda i: (i, 0)),
        out_shape=jax.ShapeDtypeStruct((TOTAL, EMBEDDING_DIM), jnp.float32),
    )(idx3, buf)


@jax.jit
def _run(x, table):
    idx_flat = x.reshape(TOTAL)
    table8 = table.reshape(NUM_EMBEDDINGS // 2, 2 * EMBEDDING_DIM)
    buf = _sc_gather(idx_flat, table8)
    out = _tc_select(idx_flat, buf)
    return out.reshape(BATCH, N_FIELDS, EMBEDDING_DIM)


def kernel(x, table):
    return _run(x, table)


# speedup vs baseline: 1.0590x; 1.0586x over previous
import jax
import jax.numpy as jnp
from jax import lax
from jax.experimental import pallas as pl
from jax.experimental.pallas import tpu as pltpu
from jax.experimental.pallas import tpu_sc as plsc

NUM_EMBEDDINGS = 16 * 1024 * 1024
EMBEDDING_DIM = 4
BATCH = 16384
N_FIELDS = 100

TOTAL = BATCH * N_FIELDS
NUM_WORKERS = 32
PER_WORKER = TOTAL // NUM_WORKERS
CHUNK = 6_400
NUM_CHUNKS = PER_WORKER // CHUNK
LANES = 16
SEL_ITERS = CHUNK * EMBEDDING_DIM // LANES


def _sc_body(idx_hbm, table8_hbm, out_hbm, idx_v, sidx_v, rows_v, sel_v, sem):
    wid = lax.axis_index("s") * 2 + lax.axis_index("c")
    base_w = wid * PER_WORKER

    lane = lax.iota(jnp.int32, LANES)
    l4 = lax.shift_right_logical(lane, 2)
    w4 = lane & 3

    def chunk_step(c, carry):
        base = base_w + c * CHUNK
        pltpu.sync_copy(idx_hbm.at[pl.ds(base, CHUNK)], idx_v)

        def shift(i, c2):
            sidx_v[pl.ds(i * LANES, LANES)] = lax.shift_right_logical(
                idx_v[pl.ds(i * LANES, LANES)], 1
            )
            return c2

        lax.fori_loop(0, CHUNK // LANES, shift, 0)
        pltpu.async_copy(table8_hbm.at[sidx_v], rows_v, sem).wait()

        def select(i, c2):
            row = i * 4 + l4
            par = plsc.load_gather(idx_v, [row]) & 1
            col = par * 4 + w4
            sel_v[pl.ds(i * LANES, LANES)] = plsc.load_gather(
                rows_v, [row, col]
            )
            return c2

        lax.fori_loop(0, SEL_ITERS, select, 0)
        pltpu.sync_copy(
            sel_v,
            out_hbm.at[pl.ds(base * EMBEDDING_DIM, CHUNK * EMBEDDING_DIM)],
        )
        return carry

    lax.fori_loop(0, NUM_CHUNKS, chunk_step, 0)


def _sc_gather(idx_flat, table8):
    mesh = plsc.VectorSubcoreMesh(core_axis_name="c", subcore_axis_name="s")
    return pl.kernel(
        _sc_body,
        out_type=jax.ShapeDtypeStruct((TOTAL * EMBEDDING_DIM,), jnp.float32),
        mesh=mesh,
        scratch_types=[
            pltpu.VMEM((CHUNK,), jnp.int32),
            pltpu.VMEM((CHUNK,), jnp.int32),
            pltpu.VMEM((CHUNK, 2 * EMBEDDING_DIM), jnp.float32),
            pltpu.VMEM((CHUNK * EMBEDDING_DIM,), jnp.float32),
            pltpu.SemaphoreType.DMA,
        ],
        compiler_params=pltpu.CompilerParams(
            use_tc_tiling_on_sc=False, needs_layout_passes=False
        ),
    )(idx_flat, table8)


@jax.jit
def _run(x, table):
    idx_flat = x.reshape(TOTAL)
    table8 = table.reshape(NUM_EMBEDDINGS // 2, 2 * EMBEDDING_DIM)
    out = _sc_gather(idx_flat, table8)
    return out.reshape(BATCH, N_FIELDS, EMBEDDING_DIM)


def kernel(x, table):
    return _run(x, table)
